# Initial kernel scaffold; baseline (speedup 1.0000x reference)
import jax
import jax.numpy as jnp
from jax.experimental import pallas as pl


def kernel(Z):
    raise NotImplementedError("write your pallas kernel here")


# speedup vs baseline: 22.4598x; 22.4598x over previous
import functools

import jax
import jax.numpy as jnp
import numpy as np
from jax.experimental import pallas as pl
from jax.experimental.pallas import tpu as pltpu

N_CELLS = 16384
N_GENES = 2048
LOGSCALE = np.float32(np.log10(10000.0))
LOGMAX = np.float32(np.log10(100000.0))
LO_P = N_CELLS // 4
HI_P = N_CELLS // 10
K2 = N_CELLS - HI_P
CB = 128
N_ITERS = 14


def _body(z_ref, zn_ref, acc_ref):
    i = pl.program_id(0)
    z = z_ref[:, :]
    zlog = jnp.log10(z)
    zn_ref[:, :] = jnp.tanh((zlog - LOGSCALE) * np.float32(1.0 / LOGSCALE))

    x = jnp.maximum(zlog, jnp.float32(0.0))

    kf1 = jnp.float32(LO_P)
    kf2 = jnp.float32(K2)

    lo0 = jnp.full((1, CB), -1.0, jnp.float32)
    hi0 = jnp.full((1, CB), 4.5, jnp.float32)

    def it(_, carry):
        lo1, hi1, lo2, hi2 = carry
        m1 = (lo1 + hi1) * 0.5
        m2 = (lo2 + hi2) * 0.5
        c1 = jnp.sum((x <= m1).astype(jnp.float32), axis=0, keepdims=True)
        c2 = jnp.sum((x <= m2).astype(jnp.float32), axis=0, keepdims=True)
        ge1 = c1 >= kf1
        ge2 = c2 >= kf2
        lo1n = jnp.where(ge1, lo1, m1)
        hi1n = jnp.where(ge1, m1, hi1)
        lo2n = jnp.where(ge2, lo2, m2)
        hi2n = jnp.where(ge2, m2, hi2)
        return lo1n, hi1n, lo2n, hi2n

    lo1, hi1, lo2, hi2 = jax.lax.fori_loop(
        0, N_ITERS, it, (lo0, hi0, lo0, hi0))

    m1f = (x <= lo1).astype(jnp.float32)
    m2f = (x <= lo2).astype(jnp.float32)
    s1 = jnp.sum(x * m1f, axis=0, keepdims=True)
    c1 = jnp.sum(m1f, axis=0, keepdims=True)
    s2 = jnp.sum(x * m2f, axis=0, keepdims=True)
    c2 = jnp.sum(m2f, axis=0, keepdims=True)
    colsum = jnp.sum(x, axis=0, keepdims=True)

    bs1 = s1 + (kf1 - c1) * (lo1 + hi1) * 0.5
    bs2 = s2 + (kf2 - c2) * (lo2 + hi2) * 0.5
    ts = colsum - bs2

    b_tot = jnp.sum(bs1)
    t_tot = jnp.sum(ts)
    row = jax.lax.broadcasted_iota(jnp.int32, (8, 128), 0)
    col = jax.lax.broadcasted_iota(jnp.int32, (8, 128), 1)
    upd = jnp.where((row == 0) & (col == 0), b_tot,
                    jnp.where((row == 0) & (col == 1), t_tot,
                              jnp.float32(0.0)))

    @pl.when(i == 0)
    def _init():
        acc_ref[:, :] = jnp.zeros((8, 128), jnp.float32)

    acc_ref[:, :] += upd


@jax.jit
def kernel(Z):
    zn, acc = pl.pallas_call(
        _body,
        grid=(N_GENES // CB,),
        in_specs=[pl.BlockSpec((N_CELLS, CB), lambda i: (0, i))],
        out_specs=[
            pl.BlockSpec((N_CELLS, CB), lambda i: (0, i)),
            pl.BlockSpec((8, 128), lambda i: (0, 0)),
        ],
        out_shape=[
            jax.ShapeDtypeStruct((N_CELLS, N_GENES), jnp.float32),
            jax.ShapeDtypeStruct((8, 128), jnp.float32),
        ],
        compiler_params=pltpu.CompilerParams(
            dimension_semantics=("arbitrary",),
        ),
    )(Z)
    lo = acc[0, 0] / np.float32(LO_P * N_GENES)
    hi = LOGMAX - acc[0, 1] / np.float32(HI_P * N_GENES)
    bit_cnst = (lo + hi).astype(jnp.float32)
    return zn, bit_cnst


# speedup vs baseline: 27.1831x; 1.2103x over previous
import functools

import jax
import jax.numpy as jnp
import numpy as np
from jax.experimental import pallas as pl
from jax.experimental.pallas import tpu as pltpu

N_CELLS = 16384
N_GENES = 2048
LOGSCALE = np.float32(np.log10(10000.0))
LOGMAX = np.float32(np.log10(100000.0))
LO_P = N_CELLS // 4
HI_P = N_CELLS // 10
K2 = N_CELLS - HI_P
CB = 128
N_ITERS = 11


def _body(z_ref, zn_ref, acc_ref):
    i = pl.program_id(0)
    z = z_ref[:, :]
    zlog = jnp.log10(z)
    zn_ref[:, :] = jnp.tanh((zlog - LOGSCALE) * np.float32(1.0 / LOGSCALE))

    x = jnp.maximum(zlog, jnp.float32(0.0))

    kf1 = jnp.float32(LO_P)
    kf2 = jnp.float32(K2)

    lo0 = jnp.full((1, CB), -1.0, jnp.float32)
    hi0 = jnp.full((1, CB), 4.5, jnp.float32)

    def it(_, carry):
        lo1, hi1, lo2, hi2 = carry
        m1 = (lo1 + hi1) * 0.5
        m2 = (lo2 + hi2) * 0.5
        c1 = jnp.sum((x <= m1).astype(jnp.float32), axis=0, keepdims=True)
        c2 = jnp.sum((x <= m2).astype(jnp.float32), axis=0, keepdims=True)
        ge1 = c1 >= kf1
        ge2 = c2 >= kf2
        lo1n = jnp.where(ge1, lo1, m1)
        hi1n = jnp.where(ge1, m1, hi1)
        lo2n = jnp.where(ge2, lo2, m2)
        hi2n = jnp.where(ge2, m2, hi2)
        return lo1n, hi1n, lo2n, hi2n

    lo1, hi1, lo2, hi2 = jax.lax.fori_loop(
        0, N_ITERS, it, (lo0, hi0, lo0, hi0))

    m1f = (x <= lo1).astype(jnp.float32)
    m2f = (x <= lo2).astype(jnp.float32)
    s1 = jnp.sum(x * m1f, axis=0, keepdims=True)
    c1 = jnp.sum(m1f, axis=0, keepdims=True)
    s2 = jnp.sum(x * m2f, axis=0, keepdims=True)
    c2 = jnp.sum(m2f, axis=0, keepdims=True)
    colsum = jnp.sum(x, axis=0, keepdims=True)

    bs1 = s1 + (kf1 - c1) * (lo1 + hi1) * 0.5
    bs2 = s2 + (kf2 - c2) * (lo2 + hi2) * 0.5
    ts = colsum - bs2

    b_tot = jnp.sum(bs1)
    t_tot = jnp.sum(ts)
    row = jax.lax.broadcasted_iota(jnp.int32, (8, 128), 0)
    col = jax.lax.broadcasted_iota(jnp.int32, (8, 128), 1)
    upd = jnp.where((row == 0) & (col == 0), b_tot,
                    jnp.where((row == 0) & (col == 1), t_tot,
                              jnp.float32(0.0)))

    @pl.when(i == 0)
    def _init():
        acc_ref[:, :] = jnp.zeros((8, 128), jnp.float32)

    acc_ref[:, :] += upd


@jax.jit
def kernel(Z):
    zn, acc = pl.pallas_call(
        _body,
        grid=(N_GENES // CB,),
        in_specs=[pl.BlockSpec((N_CELLS, CB), lambda i: (0, i))],
        out_specs=[
            pl.BlockSpec((N_CELLS, CB), lambda i: (0, i)),
            pl.BlockSpec((8, 128), lambda i: (0, 0)),
        ],
        out_shape=[
            jax.ShapeDtypeStruct((N_CELLS, N_GENES), jnp.float32),
            jax.ShapeDtypeStruct((8, 128), jnp.float32),
        ],
        compiler_params=pltpu.CompilerParams(
            dimension_semantics=("arbitrary",),
        ),
    )(Z)
    lo = acc[0, 0] / np.float32(LO_P * N_GENES)
    hi = LOGMAX - acc[0, 1] / np.float32(HI_P * N_GENES)
    bit_cnst = (lo + hi).astype(jnp.float32)
    return zn, bit_cnst


# speedup vs baseline: 44.4181x; 1.6340x over previous
import functools

import jax
import jax.numpy as jnp
import numpy as np
from jax.experimental import pallas as pl
from jax.experimental.pallas import tpu as pltpu

N_CELLS = 16384
N_GENES = 2048
LOGSCALE = np.float32(np.log10(10000.0))
LOGMAX = np.float32(np.log10(100000.0))
LO_P = N_CELLS // 4
HI_P = N_CELLS // 10
K2 = N_CELLS - HI_P
CB = 128
N_ITERS = 11


def _body(z_ref, zn_ref, acc_ref):
    i = pl.program_id(0)
    z = z_ref[:, :]
    zlog = jnp.log10(z)
    zn_ref[:, :] = jnp.tanh((zlog - LOGSCALE) * np.float32(1.0 / LOGSCALE))

    x = jnp.maximum(zlog, jnp.float32(0.0))

    kf1 = jnp.float32(LO_P)
    kf2 = jnp.float32(K2)
    nf = jnp.float32(N_CELLS)

    ones_r = jnp.ones((1, N_CELLS), jnp.bfloat16)

    def rowcount(mask_bool):
        mb = mask_bool.astype(jnp.bfloat16)
        return jax.lax.dot_general(
            ones_r, mb, (((1,), (0,)), ((), ())),
            preferred_element_type=jnp.float32)

    lo0 = jnp.full((1, CB), -1.0, jnp.float32)
    hi0 = jnp.full((1, CB), 4.5, jnp.float32)
    c0 = jnp.zeros((1, CB), jnp.float32)

    def it(_, carry):
        lo1, hi1, cl1, lo2, hi2, cl2 = carry
        m1 = (lo1 + hi1) * 0.5
        m2 = (lo2 + hi2) * 0.5
        c1 = rowcount(x <= m1)
        c2 = rowcount(x <= m2)
        ge1 = c1 >= kf1
        ge2 = c2 >= kf2
        return (jnp.where(ge1, lo1, m1), jnp.where(ge1, m1, hi1),
                jnp.where(ge1, cl1, c1),
                jnp.where(ge2, lo2, m2), jnp.where(ge2, m2, hi2),
                jnp.where(ge2, cl2, c2))

    lo1, hi1, cl1, lo2, hi2, cl2 = jax.lax.fori_loop(
        0, N_ITERS, it, (lo0, hi0, c0, lo0, hi0, c0))

    sm1 = jnp.sum(jnp.minimum(x, lo1), axis=0, keepdims=True)
    sm2 = jnp.sum(jnp.minimum(x, lo2), axis=0, keepdims=True)
    colsum = jnp.sum(x, axis=0, keepdims=True)

    s1 = sm1 - (nf - cl1) * lo1
    s2 = sm2 - (nf - cl2) * lo2
    bs1 = s1 + (kf1 - cl1) * (lo1 + hi1) * 0.5
    bs2 = s2 + (kf2 - cl2) * (lo2 + hi2) * 0.5
    ts = colsum - bs2

    b_tot = jnp.sum(bs1)
    t_tot = jnp.sum(ts)
    row = jax.lax.broadcasted_iota(jnp.int32, (8, 128), 0)
    col = jax.lax.broadcasted_iota(jnp.int32, (8, 128), 1)
    upd = jnp.where((row == 0) & (col == 0), b_tot,
                    jnp.where((row == 0) & (col == 1), t_tot,
                              jnp.float32(0.0)))

    @pl.when(i == 0)
    def _init():
        acc_ref[:, :] = jnp.zeros((8, 128), jnp.float32)

    acc_ref[:, :] += upd


@jax.jit
def kernel(Z):
    zn, acc = pl.pallas_call(
        _body,
        grid=(N_GENES // CB,),
        in_specs=[pl.BlockSpec((N_CELLS, CB), lambda i: (0, i))],
        out_specs=[
            pl.BlockSpec((N_CELLS, CB), lambda i: (0, i)),
            pl.BlockSpec((8, 128), lambda i: (0, 0)),
        ],
        out_shape=[
            jax.ShapeDtypeStruct((N_CELLS, N_GENES), jnp.float32),
            jax.ShapeDtypeStruct((8, 128), jnp.float32),
        ],
        compiler_params=pltpu.CompilerParams(
            dimension_semantics=("arbitrary",),
        ),
    )(Z)
    lo = acc[0, 0] / np.float32(LO_P * N_GENES)
    hi = LOGMAX - acc[0, 1] / np.float32(HI_P * N_GENES)
    bit_cnst = (lo + hi).astype(jnp.float32)
    return zn, bit_cnst


# speedup vs baseline: 47.4252x; 1.0677x over previous
import functools

import jax
import jax.numpy as jnp
import numpy as np
from jax.experimental import pallas as pl
from jax.experimental.pallas import tpu as pltpu

N_CELLS = 16384
N_GENES = 2048
LOGSCALE = np.float32(np.log10(10000.0))
LOGMAX = np.float32(np.log10(100000.0))
LO_P = N_CELLS // 4
HI_P = N_CELLS // 10
K2 = N_CELLS - HI_P
CB = 128
N_ITERS = 10


def _body(z_ref, zn_ref, acc_ref):
    i = pl.program_id(0)
    z = z_ref[:, :]
    zlog = jnp.log10(z)
    zn_ref[:, :] = jnp.tanh((zlog - LOGSCALE) * np.float32(1.0 / LOGSCALE))

    x = jnp.maximum(zlog, jnp.float32(0.0))

    kf1 = jnp.float32(LO_P)
    kf2 = jnp.float32(K2)
    nf = jnp.float32(N_CELLS)

    ones_r = jnp.ones((1, N_CELLS), jnp.bfloat16)

    def rowcount(mask_bool):
        mb = mask_bool.astype(jnp.bfloat16)
        return jax.lax.dot_general(
            ones_r, mb, (((1,), (0,)), ((), ())),
            preferred_element_type=jnp.float32)

    lo0 = jnp.full((1, CB), -1e-3, jnp.float32)
    hi0 = jnp.full((1, CB), 4.001, jnp.float32)
    c0 = jnp.zeros((1, CB), jnp.float32)

    def it(_, carry):
        lo1, hi1, cl1, lo2, hi2, cl2 = carry
        m1 = (lo1 + hi1) * 0.5
        m2 = (lo2 + hi2) * 0.5
        c1 = rowcount(x <= m1)
        c2 = rowcount(x <= m2)
        ge1 = c1 >= kf1
        ge2 = c2 >= kf2
        return (jnp.where(ge1, lo1, m1), jnp.where(ge1, m1, hi1),
                jnp.where(ge1, cl1, c1),
                jnp.where(ge2, lo2, m2), jnp.where(ge2, m2, hi2),
                jnp.where(ge2, cl2, c2))

    lo1, hi1, cl1, lo2, hi2, cl2 = jax.lax.fori_loop(
        0, N_ITERS, it, (lo0, hi0, c0, lo0, hi0, c0))

    sm1 = jnp.sum(jnp.minimum(x, lo1), axis=0, keepdims=True)
    sm2 = jnp.sum(jnp.minimum(x, lo2), axis=0, keepdims=True)
    colsum = jnp.sum(x, axis=0, keepdims=True)

    s1 = sm1 - (nf - cl1) * lo1
    s2 = sm2 - (nf - cl2) * lo2
    bs1 = s1 + (kf1 - cl1) * (lo1 + hi1) * 0.5
    bs2 = s2 + (kf2 - cl2) * (lo2 + hi2) * 0.5
    ts = colsum - bs2

    b_tot = jnp.sum(bs1)
    t_tot = jnp.sum(ts)
    row = jax.lax.broadcasted_iota(jnp.int32, (8, 128), 0)
    col = jax.lax.broadcasted_iota(jnp.int32, (8, 128), 1)
    upd = jnp.where((row == 0) & (col == 0), b_tot,
                    jnp.where((row == 0) & (col == 1), t_tot,
                              jnp.float32(0.0)))

    @pl.when(i == 0)
    def _init():
        acc_ref[:, :] = jnp.zeros((8, 128), jnp.float32)

    acc_ref[:, :] += upd


@jax.jit
def kernel(Z):
    zn, acc = pl.pallas_call(
        _body,
        grid=(N_GENES // CB,),
        in_specs=[pl.BlockSpec((N_CELLS, CB), lambda i: (0, i))],
        out_specs=[
            pl.BlockSpec((N_CELLS, CB), lambda i: (0, i)),
            pl.BlockSpec((8, 128), lambda i: (0, 0)),
        ],
        out_shape=[
            jax.ShapeDtypeStruct((N_CELLS, N_GENES), jnp.float32),
            jax.ShapeDtypeStruct((8, 128), jnp.float32),
        ],
        compiler_params=pltpu.CompilerParams(
            dimension_semantics=("arbitrary",),
        ),
    )(Z)
    lo = acc[0, 0] / np.float32(LO_P * N_GENES)
    hi = LOGMAX - acc[0, 1] / np.float32(HI_P * N_GENES)
    bit_cnst = (lo + hi).astype(jnp.float32)
    return zn, bit_cnst
